# speedup vs baseline: 1.8986x; 1.8986x over previous
import functools

import jax
import jax.numpy as jnp
from jax import lax
from jax.experimental import pallas as pl
from jax.experimental.pallas import tpu as pltpu
from jax.experimental.pallas import tpu_sc as plsc

_BATCH = 16384
_DIM = 128
_NUM_CORES = 2
_NUM_SUBCORES = 16
_NW = _NUM_CORES * _NUM_SUBCORES
_CHUNK = _BATCH // _NW
_S = 256
_LANES = 16

_mesh = plsc.VectorSubcoreMesh(core_axis_name="c", subcore_axis_name="s")


@functools.partial(
    pl.kernel,
    out_type=jax.ShapeDtypeStruct((_NW, _LANES), jnp.float32),
    mesh=_mesh,
    scratch_types=[
        pltpu.VMEM((_CHUNK,), jnp.int32),
        pltpu.VMEM((_S, _DIM), jnp.float32),
        pltpu.VMEM((_S, _DIM), jnp.float32),
        pltpu.VMEM((_LANES,), jnp.float32),
        pltpu.SemaphoreType.DMA,
        pltpu.SemaphoreType.DMA,
    ],
)
def _center_loss_partials(x_hbm, labels_hbm, centers_hbm, out_hbm,
                          idx_v, x_v, c_v, acc_v, sem_x, sem_c):
    wid = lax.axis_index("s") * _NUM_CORES + lax.axis_index("c")
    base = wid * _CHUNK
    pltpu.sync_copy(labels_hbm.at[pl.ds(base, _CHUNK)], idx_v)

    def sub_chunk(h, acc):
        row0 = base + h * _S
        cp_x = pltpu.async_copy(x_hbm.at[pl.ds(row0, _S)], x_v, sem_x)
        cp_c = pltpu.async_copy(
            centers_hbm.at[idx_v.at[pl.ds(h * _S, _S)]], c_v, sem_c)
        cp_x.wait()
        cp_c.wait()

        def row_body(r, a):
            for j in range(_DIM // _LANES):
                d = (x_v[r, pl.ds(j * _LANES, _LANES)]
                     - c_v[r, pl.ds(j * _LANES, _LANES)])
                a = a + d * d
            return a

        return lax.fori_loop(0, _S, row_body, acc)

    acc = lax.fori_loop(0, _CHUNK // _S, sub_chunk,
                        jnp.zeros((_LANES,), jnp.float32))
    acc_v[...] = acc
    pltpu.sync_copy(acc_v, out_hbm.at[wid])


def kernel(x, labels, centers):
    partials = _center_loss_partials(x, labels, centers)
    return jnp.sum(partials) * (0.5 / _BATCH)


# speedup vs baseline: 2.0108x; 1.0591x over previous
import functools

import jax
import jax.numpy as jnp
from jax import lax
from jax.experimental import pallas as pl
from jax.experimental.pallas import tpu as pltpu
from jax.experimental.pallas import tpu_sc as plsc

_BATCH = 16384
_DIM = 128
_NUM_CORES = 2
_NUM_SUBCORES = 16
_NW = _NUM_CORES * _NUM_SUBCORES
_CHUNK = _BATCH // _NW
_S = 128
_NSUB = _CHUNK // _S
_LANES = 16

_mesh = plsc.VectorSubcoreMesh(core_axis_name="c", subcore_axis_name="s")


@functools.partial(
    pl.kernel,
    out_type=jax.ShapeDtypeStruct((_NW, _LANES), jnp.float32),
    mesh=_mesh,
    scratch_types=[
        pltpu.VMEM((_CHUNK,), jnp.int32),
        pltpu.VMEM((2, _S, _DIM), jnp.float32),
        pltpu.VMEM((2, _S, _DIM), jnp.float32),
        pltpu.VMEM((_LANES,), jnp.float32),
        [pltpu.SemaphoreType.DMA] * 2,
        [pltpu.SemaphoreType.DMA] * 2,
    ],
)
def _center_loss_partials(x_hbm, labels_hbm, centers_hbm, out_hbm,
                          idx_v, x_v, c_v, acc_v, sems_x, sems_c):
    wid = lax.axis_index("s") * _NUM_CORES + lax.axis_index("c")
    base = wid * _CHUNK
    pltpu.sync_copy(labels_hbm.at[pl.ds(base, _CHUNK)], idx_v)

    def start(h):
        b = h % 2
        cp_x = pltpu.async_copy(
            x_hbm.at[pl.ds(base + h * _S, _S)], x_v.at[b], sems_x[b])
        cp_c = pltpu.async_copy(
            centers_hbm.at[idx_v.at[pl.ds(h * _S, _S)]], c_v.at[b],
            sems_c[b])
        return cp_x, cp_c

    inflight = start(0)
    acc = jnp.zeros((_LANES,), jnp.float32)
    for h in range(_NSUB):
        cp_x, cp_c = inflight
        if h + 1 < _NSUB:
            inflight = start(h + 1)
        cp_x.wait()
        cp_c.wait()
        b = h % 2

        def row_body(r, a):
            for j in range(_DIM // _LANES):
                d = (x_v[b, r, pl.ds(j * _LANES, _LANES)]
                     - c_v[b, r, pl.ds(j * _LANES, _LANES)])
                a = a + d * d
            return a

        acc = lax.fori_loop(0, _S, row_body, acc)

    acc_v[...] = acc
    pltpu.sync_copy(acc_v, out_hbm.at[wid])


def kernel(x, labels, centers):
    partials = _center_loss_partials(x, labels, centers)
    return jnp.sum(partials) * (0.5 / _BATCH)
